# baseline (speedup 1.0000x reference)
import functools

import jax
import jax.numpy as jnp
from jax import lax
from jax.experimental import pallas as pl
from jax.experimental.pallas import tpu as pltpu
from jax.experimental.pallas import tpu_sc as plsc

_MAX_ANS_LEN = 30
_EPS = 1e-8
_NEG = -10000.0
_SPLIT = 4
_LANES = 256
_HALO = 256
_KNEG = -1.0e15


def _stage1_kernel(idxs_ref, s0_ref, s1_ref, s2_ref, s3_ref,
                   ap_ref, bp_ref, n2p_ref, mo_ref, vm_ref):
    bi = pl.program_id(0)
    refs = (s0_ref, s1_ref, s2_ref, s3_ref)
    CS = s0_ref.shape[2]
    H = s0_ref.shape[3]
    C = _SPLIT * CS // _LANES
    sep0 = idxs_ref[bi, 0]
    sep1 = idxs_ref[bi, 1]

    q = jnp.concatenate(
        [s0_ref[0, 0, 1:2, :], s0_ref[0, 0, pl.ds(sep0 - 1, 1), :]], axis=0)

    dn = (((1,), (1,)), ((), ()))
    ones = jnp.ones((1, H), jnp.float32)
    rows_a, rows_b, rows_n = [], [], []
    for r in refs:
        chunk = r[0, 0]
        ab = lax.dot_general(q, chunk, dimension_numbers=dn,
                             preferred_element_type=jnp.float32)
        n2c = lax.dot_general(ones, chunk * chunk, dimension_numbers=dn,
                              preferred_element_type=jnp.float32)
        for j in range(CS // _LANES):
            rows_a.append(lax.slice(ab, (0, j * _LANES), (1, (j + 1) * _LANES)))
            rows_b.append(lax.slice(ab, (1, j * _LANES), (2, (j + 1) * _LANES)))
            rows_n.append(lax.slice(n2c, (0, j * _LANES), (1, (j + 1) * _LANES)))
    a2 = jnp.concatenate(rows_a, axis=0)
    b2 = jnp.concatenate(rows_b, axis=0)
    n2 = jnp.concatenate(rows_n, axis=0)

    qn = jnp.sqrt(jnp.sum(q * q))
    inv_qn = 1.0 / jnp.maximum(qn, _EPS)

    s_iota = lax.broadcasted_iota(jnp.int32, (C, _LANES), 0)
    l_iota = lax.broadcasted_iota(jnp.int32, (C, _LANES), 1)
    i_idx = s_iota * _LANES + l_iota

    ap_ref[0, 0:C, :] = a2 * inv_qn
    ap_ref[0, C:C + 1, :] = jnp.zeros((1, _LANES), jnp.float32)
    bp_ref[0, 0:C, :] = b2 * inv_qn
    bp_ref[0, C:C + 1, :] = jnp.zeros((1, _LANES), jnp.float32)
    n2p_ref[0, 0:C, :] = n2
    n2p_ref[0, C:C + 1, :] = jnp.ones((1, _LANES), jnp.float32)
    mo_ref[0] = sep1 - i_idx
    vm_ref[0] = ((i_idx > sep0) & (i_idx < sep1)).astype(jnp.int32)


def _newton_rsqrt(d):
    di = lax.bitcast_convert_type(d, jnp.int32)
    magic = jnp.full((16,), 0x5F3759DF, jnp.int32)
    y = lax.bitcast_convert_type(
        magic - lax.shift_right_logical(di, jnp.full((16,), 1, jnp.int32)),
        jnp.float32)
    half = jnp.full((16,), 0.5, jnp.float32)
    three_half = jnp.full((16,), 1.5, jnp.float32)
    for _ in range(3):
        y = y * (three_half - half * d * y * y)
    return y


def _make_sc_select(B, S):
    info = plsc.get_sparse_core_info()
    NC, NS = info.num_cores, info.num_subcores
    NW = NC * NS
    segs_per_ex = NW // B
    seg_len = S // segs_per_ex
    n_vregs = seg_len // 16
    mesh = plsc.VectorSubcoreMesh(core_axis_name="c", subcore_axis_name="s")

    @functools.partial(
        pl.kernel, mesh=mesh,
        out_type=(jax.ShapeDtypeStruct((B * S,), jnp.float32),
                  jax.ShapeDtypeStruct((B * S,), jnp.int32)),
        scratch_types=[
            pltpu.VMEM((seg_len,), jnp.float32),
            pltpu.VMEM((seg_len + 32,), jnp.float32),
            pltpu.VMEM((seg_len + 32,), jnp.float32),
            pltpu.VMEM((seg_len,), jnp.int32),
            pltpu.VMEM((seg_len,), jnp.int32),
            pltpu.VMEM((seg_len,), jnp.float32),
            pltpu.VMEM((seg_len,), jnp.int32),
        ],
    )
    def sc_select(ap_h, bp_h, n2p_h, mo_h, vm_h, mv_h, ei_h,
                  a_v, b_v, n_v, mo_v, vm_v, mvo_v, eio_v):
        wid = lax.axis_index("s") * NC + lax.axis_index("c")
        bex = wid // segs_per_ex
        seg = wid % segs_per_ex
        base = seg * seg_len
        pbase = bex * (S + _HALO) + base
        fbase = bex * S + base
        pltpu.sync_copy(ap_h.at[pl.ds(pbase, seg_len)], a_v)
        pltpu.sync_copy(bp_h.at[pl.ds(pbase, seg_len + 32)], b_v)
        pltpu.sync_copy(n2p_h.at[pl.ds(pbase, seg_len + 32)], n_v)
        pltpu.sync_copy(mo_h.at[pl.ds(fbase, seg_len)], mo_v)
        pltpu.sync_copy(vm_h.at[pl.ds(fbase, seg_len)], vm_v)

        nneg = jnp.full((16,), _KNEG, jnp.float32)
        one = jnp.full((16,), 1.0, jnp.float32)
        neg = jnp.full((16,), _NEG, jnp.float32)
        zero_i = jnp.full((16,), 0, jnp.int32)
        lane = lax.iota(jnp.int32, 16)

        for v in range(n_vregs):
            off = v * 16
            av = a_v[pl.ds(off, 16)]
            ni = n_v[pl.ds(off, 16)]
            mov = mo_v[pl.ds(off, 16)]
            vmv = vm_v[pl.ds(off, 16)]

            def body(o, carry):
                num_best, d_best, bo = carry
                bj = b_v[pl.ds(off + o, 16)]
                nj = n_v[pl.ds(off + o, 16)]
                num = av + bj
                d = ni + nj
                ob = lax.broadcast_in_dim(o, (16,), ())
                valid = mov > ob
                num = jnp.where(valid, num, nneg)
                d = jnp.where(valid, d, one)
                p = num * jnp.abs(num)
                p_best = num_best * jnp.abs(num_best)
                upd = p * d_best > p_best * d
                num_best = jnp.where(upd, num, num_best)
                d_best = jnp.where(upd, d, d_best)
                bo = jnp.where(upd, ob, bo)
                return num_best, d_best, bo

            init = (nneg, one, zero_i)
            num_best, d_best, bo = lax.fori_loop(0, _MAX_ANS_LEN, body, init)

            d = jnp.maximum(d_best, jnp.full((16,), _EPS * _EPS, jnp.float32))
            r = jnp.minimum(_newton_rsqrt(d), jnp.full((16,), 1.0 / _EPS, jnp.float32))
            val = num_best * r
            ivalid = vmv > zero_i
            gi = lane + lax.broadcast_in_dim(base + off, (16,), ())
            mvo_v[pl.ds(off, 16)] = jnp.where(ivalid, val, neg)
            eio_v[pl.ds(off, 16)] = jnp.where(ivalid, gi + bo,
                                              jnp.full((16,), -1, jnp.int32))

        pltpu.sync_copy(mvo_v, mv_h.at[pl.ds(fbase, seg_len)])
        pltpu.sync_copy(eio_v, ei_h.at[pl.ds(fbase, seg_len)])

    return sc_select


@functools.partial(jax.jit, static_argnames=())
def kernel(sequence_outputs, idxs):
    B, S, H = sequence_outputs.shape
    CS = S // _SPLIT
    C = S // _LANES
    out_shape = (
        jax.ShapeDtypeStruct((B, C + 1, _LANES), jnp.float32),
        jax.ShapeDtypeStruct((B, C + 1, _LANES), jnp.float32),
        jax.ShapeDtypeStruct((B, C + 1, _LANES), jnp.float32),
        jax.ShapeDtypeStruct((B, C, _LANES), jnp.int32),
        jax.ShapeDtypeStruct((B, C, _LANES), jnp.int32),
    )
    seq4 = sequence_outputs.reshape(B, _SPLIT, CS, H)
    specs = [
        pl.BlockSpec((1, 1, CS, H), functools.partial(
            lambda k, b: (b, k, 0, 0), k))
        for k in range(_SPLIT)
    ]
    pad_spec = pl.BlockSpec((1, C + 1, _LANES), lambda b: (b, 0, 0))
    reg_spec = pl.BlockSpec((1, C, _LANES), lambda b: (b, 0, 0))
    ap, bp, n2p, mo, vm = pl.pallas_call(
        _stage1_kernel,
        grid=(B,),
        in_specs=[pl.BlockSpec(memory_space=pltpu.SMEM)] + specs,
        out_specs=(pad_spec, pad_spec, pad_spec, reg_spec, reg_spec),
        out_shape=out_shape,
        compiler_params=pltpu.CompilerParams(
            dimension_semantics=("arbitrary",),
        ),
    )(idxs, *([seq4] * _SPLIT))

    sc_select = _make_sc_select(B, S)
    mv, ei = sc_select(ap.reshape(B * (S + _HALO)),
                       bp.reshape(B * (S + _HALO)),
                       n2p.reshape(B * (S + _HALO)),
                       mo.reshape(B * S), vm.reshape(B * S))
    return mv.reshape(B, S), ei.reshape(B, S)


# speedup vs baseline: 2.6274x; 2.6274x over previous
import functools

import jax
import jax.numpy as jnp
from jax import lax
from jax.experimental import pallas as pl
from jax.experimental.pallas import tpu as pltpu

_MAX_ANS_LEN = 30
_EPS = 1e-8
_NEG = -10000.0
_SPLIT = 4
_LANES = 256


def _matvecs(refs, idxs_ref, bex, a_s, b_s, n2_s):
    CS = refs[0].shape[2]
    H = refs[0].shape[3]
    sep0 = idxs_ref[bex, 0]
    q = jnp.concatenate(
        [refs[0][0, 0, 1:2, :], refs[0][0, 0, pl.ds(sep0 - 1, 1), :]], axis=0)
    qn = jnp.sqrt(jnp.sum(q * q))
    inv_qn = 1.0 / jnp.maximum(qn, _EPS)

    dn = (((1,), (1,)), ((), ()))
    ones = jnp.ones((1, H), jnp.float32)
    rows_a, rows_b, rows_n = [], [], []
    for r in refs:
        chunk = r[0, 0]
        ab = lax.dot_general(q, chunk, dimension_numbers=dn,
                             preferred_element_type=jnp.float32)
        n2c = lax.dot_general(ones, chunk * chunk, dimension_numbers=dn,
                              preferred_element_type=jnp.float32)
        for j in range(CS // _LANES):
            rows_a.append(lax.slice(ab, (0, j * _LANES), (1, (j + 1) * _LANES)))
            rows_b.append(lax.slice(ab, (1, j * _LANES), (2, (j + 1) * _LANES)))
            rows_n.append(lax.slice(n2c, (0, j * _LANES), (1, (j + 1) * _LANES)))
    a_s[...] = jnp.concatenate(rows_a, axis=0) * inv_qn
    b_s[...] = jnp.concatenate(rows_b, axis=0) * inv_qn
    n2_s[...] = jnp.concatenate(rows_n, axis=0)


def _windowed(idxs_ref, bex, a_s, b_s, n2_s, mv_ref, ei_ref):
    C, L = a_s.shape
    sep0 = idxs_ref[bex, 0]
    sep1 = idxs_ref[bex, 1]
    a2 = a_s[...]
    b2 = b_s[...]
    n2 = n2_s[...]
    pad_row = jnp.ones((1, L), jnp.float32)
    b_dw = jnp.concatenate(
        [b2, jnp.concatenate([b2[1:, :], pad_row], axis=0)], axis=1)
    n2_dw = jnp.concatenate(
        [n2, jnp.concatenate([n2[1:, :], pad_row], axis=0)], axis=1)

    s_iota = lax.broadcasted_iota(jnp.int32, (C, L), 0)
    l_iota = lax.broadcasted_iota(jnp.int32, (C, L), 1)
    i_idx = s_iota * L + l_iota

    mv = jnp.full((C, L), _NEG, jnp.float32)
    best_o = jnp.zeros((C, L), jnp.int32)
    for o in range(_MAX_ANS_LEN):
        b_o = lax.slice(b_dw, (0, o), (C, o + L))
        n2_o = lax.slice(n2_dw, (0, o), (C, o + L))
        num = a2 + b_o
        r = jnp.minimum(lax.rsqrt(n2 + n2_o), 1.0 / _EPS)
        sim = num * r
        valid = i_idx < (sep1 - o)
        sim = jnp.where(valid, sim, _NEG)
        if o == 0:
            mv = sim
        else:
            upd = sim > mv
            mv = jnp.where(upd, sim, mv)
            best_o = jnp.where(upd, o, best_o)

    i_valid = (i_idx > sep0) & (i_idx < sep1)
    mv_ref[0] = jnp.where(i_valid, mv, _NEG)
    ei_ref[0] = jnp.where(i_valid, i_idx + best_o, -1)


def _dcr_kernel(idxs_ref, s0_ref, s1_ref, s2_ref, s3_ref, mv_ref, ei_ref,
                a_a, b_a, n2_a, a_b, b_b, n2_b):
    s = pl.program_id(0)
    nb = pl.num_programs(0) - 1
    refs = (s0_ref, s1_ref, s2_ref, s3_ref)

    @pl.when((s < nb) & (s % 2 == 0))
    def _():
        _matvecs(refs, idxs_ref, s, a_a, b_a, n2_a)

    @pl.when((s < nb) & (s % 2 == 1))
    def _():
        _matvecs(refs, idxs_ref, s, a_b, b_b, n2_b)

    @pl.when((s > 0) & (s % 2 == 1))
    def _():
        _windowed(idxs_ref, s - 1, a_a, b_a, n2_a, mv_ref, ei_ref)

    @pl.when((s > 0) & (s % 2 == 0))
    def _():
        _windowed(idxs_ref, s - 1, a_b, b_b, n2_b, mv_ref, ei_ref)


@functools.partial(jax.jit, static_argnames=())
def kernel(sequence_outputs, idxs):
    B, S, H = sequence_outputs.shape
    CS = S // _SPLIT
    C = S // _LANES
    out_shape = (
        jax.ShapeDtypeStruct((B, C, _LANES), jnp.float32),
        jax.ShapeDtypeStruct((B, C, _LANES), jnp.int32),
    )
    seq4 = sequence_outputs.reshape(B, _SPLIT, CS, H)
    specs = [
        pl.BlockSpec((1, 1, CS, H), functools.partial(
            lambda k, s: (jnp.minimum(s, B - 1), k, 0, 0), k))
        for k in range(_SPLIT)
    ]
    out_spec = pl.BlockSpec((1, C, _LANES),
                            lambda s: (jnp.maximum(s - 1, 0), 0, 0))
    mv, ei = pl.pallas_call(
        _dcr_kernel,
        grid=(B + 1,),
        in_specs=[pl.BlockSpec(memory_space=pltpu.SMEM)] + specs,
        out_specs=(out_spec, out_spec),
        out_shape=out_shape,
        scratch_shapes=[
            pltpu.VMEM((C, _LANES), jnp.float32),
            pltpu.VMEM((C, _LANES), jnp.float32),
            pltpu.VMEM((C, _LANES), jnp.float32),
            pltpu.VMEM((C, _LANES), jnp.float32),
            pltpu.VMEM((C, _LANES), jnp.float32),
            pltpu.VMEM((C, _LANES), jnp.float32),
        ],
        compiler_params=pltpu.CompilerParams(
            dimension_semantics=("arbitrary",),
        ),
    )(idxs, *([seq4] * _SPLIT))
    return mv.reshape(B, S), ei.reshape(B, S)


# speedup vs baseline: 2.6866x; 1.0226x over previous
import functools

import jax
import jax.numpy as jnp
from jax.experimental import pallas as pl
from jax.experimental.pallas import tpu as pltpu

_MAX_ANS_LEN = 30
_EPS = 1e-8
_NEG = -10000.0
_SPLIT = 4
_LANES = 256


def _dcr_kernel(idxs_ref, s0_ref, s1_ref, s2_ref, s3_ref, mv_ref, ei_ref):
    bi = pl.program_id(0)
    refs = (s0_ref, s1_ref, s2_ref, s3_ref)
    CS = s0_ref.shape[2]
    H = s0_ref.shape[3]
    C = _SPLIT * CS // _LANES
    sep0 = idxs_ref[bi, 0]
    sep1 = idxs_ref[bi, 1]

    q = jnp.concatenate(
        [s0_ref[0, 0, 1:2, :], s0_ref[0, 0, pl.ds(sep0 - 1, 1), :]], axis=0)

    dn = (((1,), (1,)), ((), ()))
    ones = jnp.ones((1, H), jnp.float32)
    rows_a, rows_b, rows_n = [], [], []
    for r in refs:
        chunk = r[0, 0]
        ab = jax.lax.dot_general(q, chunk, dimension_numbers=dn,
                                 preferred_element_type=jnp.float32)
        n2c = jax.lax.dot_general(ones, chunk * chunk, dimension_numbers=dn,
                                  preferred_element_type=jnp.float32)
        for j in range(CS // _LANES):
            rows_a.append(jax.lax.slice(ab, (0, j * _LANES), (1, (j + 1) * _LANES)))
            rows_b.append(jax.lax.slice(ab, (1, j * _LANES), (2, (j + 1) * _LANES)))
            rows_n.append(jax.lax.slice(n2c, (0, j * _LANES), (1, (j + 1) * _LANES)))
    a2 = jnp.concatenate(rows_a, axis=0)
    b2 = jnp.concatenate(rows_b, axis=0)
    n2 = jnp.concatenate(rows_n, axis=0)

    qn = jnp.sqrt(jnp.sum(q * q))
    inv_qn = 1.0 / jnp.maximum(qn, _EPS)

    pad_row = jnp.ones((1, _LANES), jnp.float32)
    b_dw = jnp.concatenate(
        [b2, jnp.concatenate([b2[1:, :], pad_row], axis=0)], axis=1)
    n2_dw = jnp.concatenate(
        [n2, jnp.concatenate([n2[1:, :], pad_row], axis=0)], axis=1)

    s_iota = jax.lax.broadcasted_iota(jnp.int32, (C, _LANES), 0)
    l_iota = jax.lax.broadcasted_iota(jnp.int32, (C, _LANES), 1)
    i_idx = s_iota * _LANES + l_iota

    mv = jnp.full((C, _LANES), _NEG, jnp.float32)
    best_o = jnp.zeros((C, _LANES), jnp.int32)
    for o in range(_MAX_ANS_LEN):
        b_o = jax.lax.slice(b_dw, (0, o), (C, o + _LANES))
        n2_o = jax.lax.slice(n2_dw, (0, o), (C, o + _LANES))
        num = a2 + b_o
        r = jnp.minimum(jax.lax.rsqrt(n2 + n2_o), 1.0 / _EPS)
        sim = num * r * inv_qn
        valid = i_idx < (sep1 - o)
        sim = jnp.where(valid, sim, _NEG)
        if o == 0:
            mv = sim
        else:
            upd = sim > mv
            mv = jnp.where(upd, sim, mv)
            best_o = jnp.where(upd, o, best_o)

    i_valid = (i_idx > sep0) & (i_idx < sep1)
    mv_ref[0] = jnp.where(i_valid, mv, _NEG)
    ei_ref[0] = jnp.where(i_valid, i_idx + best_o, -1)


@functools.partial(jax.jit, static_argnames=())
def kernel(sequence_outputs, idxs):
    B, S, H = sequence_outputs.shape
    CS = S // _SPLIT
    C = S // _LANES
    out_shape = (
        jax.ShapeDtypeStruct((B, C, _LANES), jnp.float32),
        jax.ShapeDtypeStruct((B, C, _LANES), jnp.int32),
    )
    seq4 = sequence_outputs.reshape(B, _SPLIT, CS, H)
    specs = [
        pl.BlockSpec((1, 1, CS, H), functools.partial(
            lambda k, b: (b, k, 0, 0), k))
        for k in range(_SPLIT)
    ]
    mv, ei = pl.pallas_call(
        _dcr_kernel,
        grid=(B,),
        in_specs=[pl.BlockSpec(memory_space=pltpu.SMEM)] + specs,
        out_specs=(
            pl.BlockSpec((1, C, _LANES), lambda b: (b, 0, 0)),
            pl.BlockSpec((1, C, _LANES), lambda b: (b, 0, 0)),
        ),
        out_shape=out_shape,
        compiler_params=pltpu.CompilerParams(
            dimension_semantics=("arbitrary",),
        ),
    )(idxs, *([seq4] * _SPLIT))
    return mv.reshape(B, S), ei.reshape(B, S)
